# baseline (speedup 1.0000x reference)
import functools

import jax
import jax.numpy as jnp
from jax import lax
from jax.experimental import pallas as pl
from jax.experimental.pallas import tpu as pltpu
from jax.experimental.pallas import tpu_sc as plsc

_E = 64
_TOPK = 8
_D = 4096
_S = 8192
_BS = 1024
_NB = _S // _BS
_CS = 256
_NC = _BS // _CS
_EPS = float(jnp.finfo(jnp.float32).eps)

_SC_CORES = 2
_SC_SUBCORES = 16
_SC_LANES = 16
_NW = _SC_CORES * _SC_SUBCORES
_CHW = _S // _NW
_GW = 128


def _router_body(x_ref, w_ref, triu_ref, *refs):
    gates_refs = refs[0:_TOPK]
    idx_refs = refs[_TOPK:2 * _TOPK]
    locw_refs = refs[2 * _TOPK:3 * _TOPK]
    off_ref, loss_ref, carry_ref, me_acc = refs[3 * _TOPK:]
    b = pl.program_id(0)

    @pl.when(b == 0)
    def _():
        carry_ref[...] = jnp.zeros_like(carry_ref)
        me_acc[...] = jnp.zeros_like(me_acc)

    logits = jax.lax.dot_general(
        w_ref[...], x_ref[...], (((1,), (1,)), ((), ())),
        preferred_element_type=jnp.float32)

    iota_e = jax.lax.broadcasted_iota(jnp.int32, (_E, _BS), 0)

    cur = logits
    val_rows, idx_rows, sels = [], [], []
    for _ in range(_TOPK):
        m = jnp.max(cur, axis=0, keepdims=True)
        ik = jnp.min(jnp.where(cur == m, iota_e, _E), axis=0, keepdims=True)
        sel = iota_e == ik
        cur = jnp.where(sel, -jnp.inf, cur)
        val_rows.append(m)
        idx_rows.append(ik)
        sels.append(sel)

    maxv = val_rows[0]
    expl = jnp.exp(logits - maxv)
    inv = 1.0 / jnp.sum(expl, axis=0, keepdims=True)
    gate_rows = [jnp.exp(v - maxv) * inv for v in val_rows]
    denom = gate_rows[0]
    for g in gate_rows[1:]:
        denom = denom + g
    inv_denom = 1.0 / jnp.maximum(denom, _EPS)
    for k in range(_TOPK):
        gates_refs[k][...] = gate_rows[k] * inv_denom
        idx_refs[k][...] = idx_rows[k]

    sel_all = jnp.concatenate(sels, axis=0).astype(jnp.bfloat16)
    csum_chunks = []
    off = None
    for c in range(_NC):
        part = jax.lax.dot_general(
            sel_all[:, c * _CS:(c + 1) * _CS], triu_ref[...],
            (((1,), (0,)), ((), ())), preferred_element_type=jnp.float32)
        if off is not None:
            part = part + off
        off = part[:, _CS - 1:_CS]
        csum_chunks.append(part)
    csum_all = jnp.concatenate(csum_chunks, axis=1)
    cnt_cols = []
    for k in range(_TOPK):
        sel = sels[k]
        csum = csum_all[k * _E:(k + 1) * _E, :]
        carry_k = carry_ref[:, k:k + 1]
        loc_f = jnp.sum(jnp.where(sel, csum - 1.0 + carry_k, 0.0),
                        axis=0, keepdims=True)
        locw_refs[k][...] = loc_f.astype(jnp.int32)
        cnt_cols.append(csum[:, _BS - 1:_BS])
    carry_ref[...] = carry_ref[...] + jnp.concatenate(cnt_cols, axis=1)

    me_acc[...] = me_acc[...] + jnp.sum(expl * inv, axis=1, keepdims=True)

    @pl.when(b == _NB - 1)
    def _():
        carry = carry_ref[...]
        off_cols = [jnp.zeros((_E, 1), jnp.float32)]
        acc = carry[:, 0:1]
        for k in range(1, _TOPK):
            off_cols.append(acc)
            if k < _TOPK - 1:
                acc = acc + carry[:, k:k + 1]
        off_ref[...] = jnp.concatenate(off_cols, axis=1).astype(jnp.int32)
        loss_ref[...] = (jnp.sum(me_acc[...] * carry[:, 0:1],
                                 keepdims=True) * (_E / (_S * _S)))


def _make_sc_offset():
    mesh = plsc.VectorSubcoreMesh(core_axis_name="c", subcore_axis_name="s")
    n = _TOPK * _CHW

    @functools.partial(
        pl.kernel,
        out_type=[jax.ShapeDtypeStruct((_S,), jnp.int32)] * _TOPK,
        mesh=mesh,
        scratch_types=[
            pltpu.VMEM((n,), jnp.int32),
            pltpu.VMEM((n,), jnp.int32),
            pltpu.VMEM((n,), jnp.int32),
            pltpu.VMEM((n,), jnp.int32),
            pltpu.VMEM((n,), jnp.int32),
            pltpu.SemaphoreType.DMA,
        ],
    )
    def sc_offset(off_hbm, *rest):
        locw_hbm = rest[0:_TOPK]
        idx_hbm = rest[_TOPK:2 * _TOPK]
        out_hbm = rest[2 * _TOPK:3 * _TOPK]
        idx_all, locw_all, flat_all, ov_all, out_all = rest[3 * _TOPK:-1]
        sem = rest[-1]
        wid = lax.axis_index("s") * _SC_CORES + lax.axis_index("c")
        span = pl.ds(wid * _CHW, _CHW)
        hs = []
        for k in range(_TOPK):
            ksl = pl.ds(k * _CHW, _CHW)
            hs.append(pltpu.async_copy(idx_hbm[k].at[span],
                                       idx_all.at[ksl], sem))
            hs.append(pltpu.async_copy(locw_hbm[k].at[span],
                                       locw_all.at[ksl], sem))
        for h in hs:
            h.wait()
        for g in range(n // _SC_LANES):
            k = g // (_CHW // _SC_LANES)
            sl = pl.ds(g * _SC_LANES, _SC_LANES)
            flat_all[sl] = idx_all[sl] * _TOPK + k
        hs = []
        for h in range(n // _GW):
            gsl = pl.ds(h * _GW, _GW)
            hs.append(pltpu.async_copy(off_hbm.at[flat_all.at[gsl]],
                                       ov_all.at[gsl], sem))
        for h in hs:
            h.wait()
        for g in range(n // _SC_LANES):
            sl = pl.ds(g * _SC_LANES, _SC_LANES)
            out_all[sl] = locw_all[sl] + ov_all[sl]
        hs = []
        for k in range(_TOPK):
            ksl = pl.ds(k * _CHW, _CHW)
            hs.append(pltpu.async_copy(out_all.at[ksl],
                                       out_hbm[k].at[span], sem))
        for h in hs:
            h.wait()

    return sc_offset


def _run(x, W, interpret=False):
    triu = jnp.triu(jnp.ones((_CS, _CS), jnp.bfloat16))
    row_spec = pl.BlockSpec((1, _BS), lambda i: (0, i))
    outs = pl.pallas_call(
        _router_body,
        grid=(_NB,),
        in_specs=[
            pl.BlockSpec((_BS, _D), lambda i: (i, 0)),
            pl.BlockSpec((_E, _D), lambda i: (0, 0)),
            pl.BlockSpec((_CS, _CS), lambda i: (0, 0)),
        ],
        out_specs=(
            [row_spec] * (3 * _TOPK)
            + [pl.BlockSpec((_E, _TOPK), lambda i: (0, 0)),
               pl.BlockSpec((1, 1), lambda i: (0, 0))]
        ),
        out_shape=(
            [jax.ShapeDtypeStruct((1, _S), jnp.float32)] * _TOPK
            + [jax.ShapeDtypeStruct((1, _S), jnp.int32)] * (2 * _TOPK)
            + [jax.ShapeDtypeStruct((_E, _TOPK), jnp.int32),
               jax.ShapeDtypeStruct((1, 1), jnp.float32)]
        ),
        scratch_shapes=[
            pltpu.VMEM((_E, _TOPK), jnp.float32),
            pltpu.VMEM((_E, 1), jnp.float32),
        ],
        interpret=interpret,
    )(x, W, triu)
    gates = outs[0:_TOPK]
    idxs = outs[_TOPK:2 * _TOPK]
    locws = outs[2 * _TOPK:3 * _TOPK]
    off, loss = outs[3 * _TOPK:]
    locs = _make_sc_offset()(
        jnp.reshape(off, (_E * _TOPK,)),
        *[jnp.reshape(l, (_S,)) for l in locws],
        *[jnp.reshape(i, (_S,)) for i in idxs])
    return gates, idxs, locs, loss


def kernel(input, W):
    gates, idxs, locs, loss = _run(input, W)
    return (jnp.reshape(loss, ()),
            tuple(jnp.reshape(g, (_S,)) for g in gates),
            tuple(jnp.reshape(i, (_S,)) for i in idxs),
            tuple(jnp.reshape(l, (_S,)) for l in locs))


# speedup vs baseline: 2.1414x; 2.1414x over previous
import functools

import jax
import jax.numpy as jnp
from jax import lax
from jax.experimental import pallas as pl
from jax.experimental.pallas import tpu as pltpu
from jax.experimental.pallas import tpu_sc as plsc

_E = 64
_TOPK = 8
_D = 4096
_S = 8192
_BS = 1024
_NB = _S // _BS
_CS = 256
_NC = _BS // _CS
_EPS = float(jnp.finfo(jnp.float32).eps)

_SC_CORES = 2
_SC_SUBCORES = 16
_SC_LANES = 16
_NW = _SC_CORES * _SC_SUBCORES
_CHW = _S // _NW
_GW = 128


def _router_body(x_ref, w_ref, triu_ref, *refs):
    gates_refs = refs[0:_TOPK]
    idx_refs = refs[_TOPK:2 * _TOPK]
    locw_refs = refs[2 * _TOPK:3 * _TOPK]
    off_ref, loss_ref, carry_ref, me_acc, cnt_lanes = refs[3 * _TOPK:]
    b = pl.program_id(0)

    @pl.when(b == 0)
    def _():
        carry_ref[...] = jnp.zeros_like(carry_ref)
        me_acc[...] = jnp.zeros_like(me_acc)
        cnt_lanes[...] = jnp.zeros_like(cnt_lanes)

    logits = jax.lax.dot_general(
        w_ref[...], x_ref[...], (((1,), (1,)), ((), ())),
        preferred_element_type=jnp.float32)

    iota_e = jax.lax.broadcasted_iota(jnp.int32, (_E, _BS), 0)

    cur = logits
    val_rows, idx_rows, sels = [], [], []
    for _ in range(_TOPK):
        m = jnp.max(cur, axis=0, keepdims=True)
        ik = jnp.min(jnp.where(cur == m, iota_e, _E), axis=0, keepdims=True)
        sel = iota_e == ik
        cur = jnp.where(sel, -jnp.inf, cur)
        val_rows.append(m)
        idx_rows.append(ik)
        sels.append(sel)

    maxv = val_rows[0]
    expl = jnp.exp(logits - maxv)
    inv = 1.0 / jnp.sum(expl, axis=0, keepdims=True)
    gate_rows = [jnp.exp(v - maxv) * inv for v in val_rows]
    denom = gate_rows[0]
    for g in gate_rows[1:]:
        denom = denom + g
    inv_denom = 1.0 / jnp.maximum(denom, _EPS)
    for k in range(_TOPK):
        gates_refs[k][...] = gate_rows[k] * inv_denom
        idx_refs[k][...] = idx_rows[k]

    sel_all = jnp.concatenate(sels, axis=0).astype(jnp.bfloat16)
    csum_chunks = []
    off = None
    for c in range(_NC):
        part = jax.lax.dot_general(
            sel_all[:, c * _CS:(c + 1) * _CS], triu_ref[...],
            (((1,), (0,)), ((), ())), preferred_element_type=jnp.float32)
        if off is not None:
            part = part + off
        off = part[:, _CS - 1:_CS]
        csum_chunks.append(part)
    csum_all = jnp.concatenate(csum_chunks, axis=1)
    cnt_cols = []
    for k in range(_TOPK):
        sel = sels[k]
        csum = csum_all[k * _E:(k + 1) * _E, :]
        carry_k = carry_ref[:, k:k + 1]
        loc_f = jnp.sum(jnp.where(sel, csum - 1.0 + carry_k, 0.0),
                        axis=0, keepdims=True)
        locw_refs[k][...] = loc_f.astype(jnp.int32)
        cnt_cols.append(csum[:, _BS - 1:_BS])
    carry_ref[...] = carry_ref[...] + jnp.concatenate(cnt_cols, axis=1)
    cnt_lanes[...] = cnt_lanes[...] + jax.lax.dot_general(
        jnp.ones((1, _BS), jnp.bfloat16), sel_all, (((1,), (1,)), ((), ())),
        preferred_element_type=jnp.float32)

    me_acc[...] = me_acc[...] + jnp.sum(expl * inv, axis=1, keepdims=True)

    @pl.when(b == _NB - 1)
    def _():
        cnt = cnt_lanes[...]
        parts = [jnp.zeros((1, _E), jnp.float32)]
        acc = cnt[:, 0:_E]
        for k in range(1, _TOPK):
            parts.append(acc)
            if k < _TOPK - 1:
                acc = acc + cnt[:, k * _E:(k + 1) * _E]
        off_ref[...] = jnp.concatenate(parts, axis=1).astype(jnp.int32)
        loss_ref[...] = (jnp.sum(me_acc[...] * carry_ref[:, 0:1],
                                 keepdims=True) * (_E / (_S * _S)))


def _make_sc_offset():
    mesh = plsc.VectorSubcoreMesh(core_axis_name="c", subcore_axis_name="s")
    n = _TOPK * _CHW

    @functools.partial(
        pl.kernel,
        out_type=[jax.ShapeDtypeStruct((_S,), jnp.int32)] * _TOPK,
        mesh=mesh,
        scratch_types=[
            pltpu.VMEM((_E * _TOPK,), jnp.int32),
            pltpu.VMEM((n,), jnp.int32),
            pltpu.VMEM((n,), jnp.int32),
            pltpu.VMEM((n,), jnp.int32),
            pltpu.SemaphoreType.DMA,
        ],
    )
    def sc_offset(off_hbm, *rest):
        locw_hbm = rest[0:_TOPK]
        idx_hbm = rest[_TOPK:2 * _TOPK]
        out_hbm = rest[2 * _TOPK:3 * _TOPK]
        off_v, idx_all, locw_all, out_all = rest[3 * _TOPK:-1]
        sem = rest[-1]
        wid = lax.axis_index("s") * _SC_CORES + lax.axis_index("c")
        span = pl.ds(wid * _CHW, _CHW)
        pltpu.sync_copy(off_hbm, off_v)
        hs = []
        for k in range(_TOPK):
            ksl = pl.ds(k * _CHW, _CHW)
            hs.append(pltpu.async_copy(idx_hbm[k].at[span],
                                       idx_all.at[ksl], sem))
            hs.append(pltpu.async_copy(locw_hbm[k].at[span],
                                       locw_all.at[ksl], sem))
        for h in hs:
            h.wait()
        dn = lax.GatherDimensionNumbers(
            offset_dims=(), collapsed_slice_dims=(0,), start_index_map=(0,))
        for k in range(_TOPK):
            tbl = [off_v[pl.ds(k * _E + c * _SC_LANES, _SC_LANES)]
                   for c in range(_E // _SC_LANES)]
            for g in range(_CHW // _SC_LANES):
                sl = pl.ds(k * _CHW + g * _SC_LANES, _SC_LANES)
                iv = idx_all[sl]
                low = (iv & (_SC_LANES - 1))[:, None]
                hi = iv >> 4
                res = lax.gather(
                    tbl[0], low, dn, (1,),
                    mode=lax.GatherScatterMode.PROMISE_IN_BOUNDS)
                for c in range(1, _E // _SC_LANES):
                    part = lax.gather(
                        tbl[c], low, dn, (1,),
                        mode=lax.GatherScatterMode.PROMISE_IN_BOUNDS)
                    res = jnp.where(hi == c, part, res)
                out_all[sl] = locw_all[sl] + res
        hs = []
        for k in range(_TOPK):
            ksl = pl.ds(k * _CHW, _CHW)
            hs.append(pltpu.async_copy(out_all.at[ksl],
                                       out_hbm[k].at[span], sem))
        for h in hs:
            h.wait()

    return sc_offset


def _run(x, W, interpret=False):
    triu = jnp.triu(jnp.ones((_CS, _CS), jnp.bfloat16))
    row_spec = pl.BlockSpec((1, _BS), lambda i: (0, i))
    outs = pl.pallas_call(
        _router_body,
        grid=(_NB,),
        in_specs=[
            pl.BlockSpec((_BS, _D), lambda i: (i, 0)),
            pl.BlockSpec((_E, _D), lambda i: (0, 0)),
            pl.BlockSpec((_CS, _CS), lambda i: (0, 0)),
        ],
        out_specs=(
            [row_spec] * (3 * _TOPK)
            + [pl.BlockSpec((1, _E * _TOPK), lambda i: (0, 0)),
               pl.BlockSpec((1, 1), lambda i: (0, 0))]
        ),
        out_shape=(
            [jax.ShapeDtypeStruct((1, _S), jnp.float32)] * _TOPK
            + [jax.ShapeDtypeStruct((1, _S), jnp.int32)] * (2 * _TOPK)
            + [jax.ShapeDtypeStruct((1, _E * _TOPK), jnp.int32),
               jax.ShapeDtypeStruct((1, 1), jnp.float32)]
        ),
        scratch_shapes=[
            pltpu.VMEM((_E, _TOPK), jnp.float32),
            pltpu.VMEM((_E, 1), jnp.float32),
            pltpu.VMEM((1, _E * _TOPK), jnp.float32),
        ],
        interpret=interpret,
    )(x, W, triu)
    gates = outs[0:_TOPK]
    idxs = outs[_TOPK:2 * _TOPK]
    locws = outs[2 * _TOPK:3 * _TOPK]
    off, loss = outs[3 * _TOPK:]
    locs = _make_sc_offset()(
        jnp.reshape(off, (_E * _TOPK,)),
        *[jnp.reshape(l, (_S,)) for l in locws],
        *[jnp.reshape(i, (_S,)) for i in idxs])
    return gates, idxs, locs, loss


def kernel(input, W):
    gates, idxs, locs, loss = _run(input, W)
    return (jnp.reshape(loss, ()),
            tuple(jnp.reshape(g, (_S,)) for g in gates),
            tuple(jnp.reshape(i, (_S,)) for i in idxs),
            tuple(jnp.reshape(l, (_S,)) for l in locs))
